# baseline (speedup 1.0000x reference)
import functools

import jax
import jax.numpy as jnp
from jax import lax
from jax.experimental import pallas as pl
from jax.experimental.pallas import tpu as pltpu
from jax.experimental.pallas import tpu_sc as plsc

N = 10000
D = 128
E = 320000

NC = 2
NS = 16
C = 128
NPAD = 10240
ROWS_PER_TILE = NPAD // NS

EPAD = 327680
NCHUNK = EPAD // C
HALF = NCHUNK // NC
CPT = HALF // NS
CPT2 = CPT // 2

_MESH = plsc.VectorSubcoreMesh(core_axis_name="c", subcore_axis_name="s")


def _cnt_body(dst_hbm, out_hbm, cnt_sp, zbuf, ones_v, didx_all, csem):
    c = lax.axis_index("c")
    s = lax.axis_index("s")

    zeros16 = jnp.zeros((16,), jnp.float32)
    ones16 = jnp.ones((16,), jnp.float32)

    def _zb(i, _):
        zbuf[pl.ds(i * 16, 16)] = zeros16
        return 0

    lax.fori_loop(0, 2048 // 16, _zb, 0)

    def _ob(i, _):
        ones_v[pl.ds(i * 16, 16)] = ones16
        return 0

    lax.fori_loop(0, C // 16, _ob, 0)

    @pl.when(s < 5)
    def _():
        pltpu.sync_copy(zbuf, cnt_sp.at[pl.ds(s * 2048, 2048)])

    t0 = c * HALF + s * CPT
    pltpu.sync_copy(dst_hbm.at[pl.ds(t0, CPT)], didx_all)

    plsc.subcore_barrier()

    def _fire(k, _):
        pltpu.async_copy(ones_v, cnt_sp.at[didx_all.at[k]], csem, add=True)
        return 0

    lax.fori_loop(0, CPT, _fire, 0)

    def _drain(k, _):
        pltpu.make_async_copy(ones_v, cnt_sp.at[pl.ds(0, C)], csem).wait()
        return 0

    lax.fori_loop(0, CPT, _drain, 0)

    plsc.subcore_barrier()

    @pl.when(s < 10)
    def _():
        pltpu.sync_copy(cnt_sp.at[pl.ds(s * 1000, 1000)],
                        zbuf.at[pl.ds(0, 1000)])
        pltpu.sync_copy(zbuf.at[pl.ds(0, 1000)],
                        out_hbm.at[pl.ds(c * N + s * 1000, 1000)])


@functools.partial(
    pl.kernel,
    out_type=jax.ShapeDtypeStruct((NC * N,), jnp.float32),
    mesh=_MESH,
    scratch_types=[
        pltpu.VMEM_SHARED((NPAD,), jnp.float32),
        pltpu.VMEM((2048,), jnp.float32),
        pltpu.VMEM((C,), jnp.float32),
        pltpu.VMEM((CPT, C), jnp.int32),
        pltpu.SemaphoreType.DMA,
    ],
    name="sc_degree_count",
)
def _sc_cnt(dst_hbm, out_hbm, cnt_sp, zbuf, ones_v, didx_all, csem):
    _cnt_body(dst_hbm, out_hbm, cnt_sp, zbuf, ones_v, didx_all, csem)


def _edge_body(hs_hbm, src_hbm, dst_hbm, out_hbm, acc_sp, r0, r1,
               sidx_all, didx_all, g0, g1, s0, s1):
    c = lax.axis_index("c")
    s = lax.axis_index("s")
    rows = (r0, r1)
    gsem = (g0, g1)
    ssem = (s0, s1)

    zeros16 = jnp.zeros((16,), jnp.float32)

    def _zb(i, _):
        r0[i // 8, pl.ds((i % 8) * 16, 16)] = zeros16
        return 0

    lax.fori_loop(0, C * D // 16, _zb, 0)

    def _zc(k, _):
        pltpu.sync_copy(r0, acc_sp.at[pl.ds(s * ROWS_PER_TILE + k * C, C)])
        return 0

    lax.fori_loop(0, ROWS_PER_TILE // C, _zc, 0)

    plsc.subcore_barrier()

    for h in range(2):
        t0 = c * HALF + s * CPT + h * CPT2
        pltpu.sync_copy(src_hbm.at[pl.ds(t0, CPT2)], sidx_all)
        pltpu.sync_copy(dst_hbm.at[pl.ds(t0, CPT2)], didx_all)

        pltpu.async_copy(hs_hbm.at[sidx_all.at[0]], rows[0], gsem[0])

        def _grp(g, _):
            for b in range(2):
                k = g * 2 + b
                bn = 1 - b

                @pl.when(k > 0)
                def _():
                    pltpu.make_async_copy(rows[bn], acc_sp.at[pl.ds(0, C)],
                                          ssem[bn]).wait()

                @pl.when(k + 1 < CPT2)
                def _():
                    pltpu.async_copy(hs_hbm.at[sidx_all.at[k + 1]],
                                     rows[bn], gsem[bn])

                pltpu.make_async_copy(hs_hbm.at[pl.ds(0, C)], rows[b],
                                      gsem[b]).wait()
                pltpu.async_copy(rows[b], acc_sp.at[didx_all.at[k]],
                                 ssem[b], add=True)
            return 0

        lax.fori_loop(0, CPT2 // 2, _grp, 0)

        pltpu.make_async_copy(rows[1], acc_sp.at[pl.ds(0, C)],
                              ssem[1]).wait()

    plsc.subcore_barrier()

    def _wb(k, _):
        r0_ = s * ROWS_PER_TILE + k * C
        pltpu.sync_copy(acc_sp.at[pl.ds(r0_, C)], r0)
        pltpu.sync_copy(r0, out_hbm.at[pl.ds(c * NPAD + r0_, C)])
        return 0

    lax.fori_loop(0, ROWS_PER_TILE // C, _wb, 0)


@functools.partial(
    pl.kernel,
    out_type=jax.ShapeDtypeStruct((NC * NPAD, D), jnp.float32),
    mesh=_MESH,
    scratch_types=[
        pltpu.VMEM_SHARED((NPAD, D), jnp.float32),
        pltpu.VMEM((C, D), jnp.float32),
        pltpu.VMEM((C, D), jnp.float32),
        pltpu.VMEM((CPT2, C), jnp.int32),
        pltpu.VMEM((CPT2, C), jnp.int32),
        pltpu.SemaphoreType.DMA,
        pltpu.SemaphoreType.DMA,
        pltpu.SemaphoreType.DMA,
        pltpu.SemaphoreType.DMA,
    ],
    name="sc_edge_scatter",
)
def _sc_edge(hs_hbm, src_hbm, dst_hbm, out_hbm, acc_sp, r0, r1,
             sidx_all, didx_all, g0, g1, s0, s1):
    _edge_body(hs_hbm, src_hbm, dst_hbm, out_hbm, acc_sp, r0, r1,
               sidx_all, didx_all, g0, g1, s0, s1)


def _tc_mm_body(x_ref, w_ref, o_ref):
    o_ref[...] = jnp.dot(x_ref[...], w_ref[...],
                         preferred_element_type=jnp.float32)


def _tc_mm(x, w):
    return pl.pallas_call(
        _tc_mm_body,
        out_shape=jax.ShapeDtypeStruct((x.shape[0], w.shape[1]), jnp.float32),
    )(x, w)


def _dinv_of(cnt_ref):
    deg = 1.0 + cnt_ref[0, :] + cnt_ref[1, :]
    return lax.rsqrt(deg)[:, None]


def _tc_scale_body(h_ref, cnt_ref, hs_ref):
    hs_ref[...] = h_ref[...] * _dinv_of(cnt_ref)


def _tc_scale(h, cnt2):
    return pl.pallas_call(
        _tc_scale_body,
        out_shape=jax.ShapeDtypeStruct((N, D), jnp.float32),
    )(h, cnt2)


def _tc_mid_body(acc_ref, hs_ref, cnt_ref, b_ref, w_ref, o_ref):
    dinv = _dinv_of(cnt_ref)
    g = dinv * (acc_ref[0:N, :] + acc_ref[NPAD:NPAD + N, :] + hs_ref[...]) \
        + b_ref[...]
    z = jnp.maximum(g, 0.0)
    h2 = jnp.dot(z, w_ref[...], preferred_element_type=jnp.float32)
    o_ref[...] = h2 * dinv


def _tc_mid(acc, hs1, cnt2, b1, w2):
    return pl.pallas_call(
        _tc_mid_body,
        out_shape=jax.ShapeDtypeStruct((N, D), jnp.float32),
    )(acc, hs1, cnt2, b1, w2)


def _tc_fin_body(acc_ref, hs_ref, cnt_ref, b_ref, wv_ref, bv_ref, wt_ref,
                 bt_ref, h_ref, xv_ref, xt_ref):
    dinv = _dinv_of(cnt_ref)
    h = dinv * (acc_ref[0:N, :] + acc_ref[NPAD:NPAD + N, :] + hs_ref[...]) \
        + b_ref[...]
    h_ref[...] = h
    xv_ref[...] = jnp.maximum(
        jnp.dot(h, wv_ref[...], preferred_element_type=jnp.float32)
        + bv_ref[...], 0.0)
    xt_ref[...] = jnp.maximum(
        jnp.dot(h, wt_ref[...], preferred_element_type=jnp.float32)
        + bt_ref[...], 0.0)


def _tc_fin(acc, hs2, cnt2, b2, wv, bv, wt, bt):
    return pl.pallas_call(
        _tc_fin_body,
        out_shape=(
            jax.ShapeDtypeStruct((N, D), jnp.float32),
            jax.ShapeDtypeStruct((N, D), jnp.float32),
            jax.ShapeDtypeStruct((N, D), jnp.float32),
        ),
    )(acc, hs2, cnt2, b2, wv, bv, wt, bt)


def kernel(x, edge_index, W1, b1, W2, b2, Wv, bv, Wt, bt):
    src = edge_index[0]
    dst = edge_index[1]
    npadded = EPAD - E
    pad_src = jnp.arange(npadded, dtype=jnp.int32) % N
    pad_dst = N + (jnp.arange(npadded, dtype=jnp.int32) % (NPAD - N))
    src_p = jnp.concatenate([src, pad_src])
    dst_p = jnp.concatenate([dst, pad_dst])
    src2 = src_p.reshape(NCHUNK, C)
    dst2 = dst_p.reshape(NCHUNK, C)
    b1r = b1.reshape(1, D)
    b2r = b2.reshape(1, D)
    bvr = bv.reshape(1, D)
    btr = bt.reshape(1, D)

    cnt2 = _sc_cnt(dst2).reshape(NC, N)
    h1 = _tc_mm(x, W1)
    hs1 = _tc_scale(h1, cnt2)
    acc1 = _sc_edge(hs1, src2, dst2)
    hs2 = _tc_mid(acc1, hs1, cnt2, b1r, W2)
    acc2 = _sc_edge(hs2, src2, dst2)
    h, xv, xt = _tc_fin(acc2, hs2, cnt2, b2r, Wv, bvr, Wt, btr)
    return (h, xv, xt)


# speedup vs baseline: 1.0033x; 1.0033x over previous
import functools

import jax
import jax.numpy as jnp
from jax import lax
from jax.experimental import pallas as pl
from jax.experimental.pallas import tpu as pltpu
from jax.experimental.pallas import tpu_sc as plsc

N = 10000
D = 128
E = 320000

NC = 2
NS = 16
C = 128
NPAD = 10240
ROWS_PER_TILE = NPAD // NS

EPAD = 327680
NCHUNK = EPAD // C
HALF = NCHUNK // NC
CPT = HALF // NS
CPT2 = CPT // 2

_MESH = plsc.VectorSubcoreMesh(core_axis_name="c", subcore_axis_name="s")


def _cnt_body(dst_hbm, out_hbm, cnt_sp, zbuf, ones_v, didx_all, csem):
    c = lax.axis_index("c")
    s = lax.axis_index("s")

    zeros16 = jnp.zeros((16,), jnp.float32)
    ones16 = jnp.ones((16,), jnp.float32)

    def _zb(i, _):
        zbuf[pl.ds(i * 16, 16)] = zeros16
        return 0

    lax.fori_loop(0, 2048 // 16, _zb, 0)

    def _ob(i, _):
        ones_v[pl.ds(i * 16, 16)] = ones16
        return 0

    lax.fori_loop(0, C // 16, _ob, 0)

    @pl.when(s < 5)
    def _():
        pltpu.sync_copy(zbuf, cnt_sp.at[pl.ds(s * 2048, 2048)])

    t0 = c * HALF + s * CPT
    pltpu.sync_copy(dst_hbm.at[pl.ds(t0, CPT)], didx_all)

    plsc.subcore_barrier()

    def _fire(k, _):
        pltpu.async_copy(ones_v, cnt_sp.at[didx_all.at[k]], csem, add=True)
        return 0

    lax.fori_loop(0, CPT, _fire, 0)

    def _drain(k, _):
        pltpu.make_async_copy(ones_v, cnt_sp.at[pl.ds(0, C)], csem).wait()
        return 0

    lax.fori_loop(0, CPT, _drain, 0)

    plsc.subcore_barrier()

    @pl.when(s < 10)
    def _():
        pltpu.sync_copy(cnt_sp.at[pl.ds(s * 1000, 1000)],
                        zbuf.at[pl.ds(0, 1000)])
        pltpu.sync_copy(zbuf.at[pl.ds(0, 1000)],
                        out_hbm.at[pl.ds(c * N + s * 1000, 1000)])


@functools.partial(
    pl.kernel,
    out_type=jax.ShapeDtypeStruct((NC * N,), jnp.float32),
    mesh=_MESH,
    scratch_types=[
        pltpu.VMEM_SHARED((NPAD,), jnp.float32),
        pltpu.VMEM((2048,), jnp.float32),
        pltpu.VMEM((C,), jnp.float32),
        pltpu.VMEM((CPT, C), jnp.int32),
        pltpu.SemaphoreType.DMA,
    ],
    name="sc_degree_count",
)
def _sc_cnt(dst_hbm, out_hbm, cnt_sp, zbuf, ones_v, didx_all, csem):
    _cnt_body(dst_hbm, out_hbm, cnt_sp, zbuf, ones_v, didx_all, csem)


def _edge_body(hs_hbm, src_hbm, dst_hbm, out_hbm, acc_sp, r0, r1,
               sidx_all, didx_all, g0, g1, s0, s1):
    c = lax.axis_index("c")
    s = lax.axis_index("s")
    rows = (r0, r1)
    gsem = (g0, g1)
    ssem = (s0, s1)

    zeros16 = jnp.zeros((16,), jnp.float32)

    def _zb(i, _):
        r0[i // 8, pl.ds((i % 8) * 16, 16)] = zeros16
        return 0

    lax.fori_loop(0, C * D // 16, _zb, 0)

    def _zc(k, _):
        pltpu.sync_copy(r0, acc_sp.at[pl.ds(s * ROWS_PER_TILE + k * C, C)])
        return 0

    lax.fori_loop(0, ROWS_PER_TILE // C, _zc, 0)

    plsc.subcore_barrier()

    for h in range(2):
        t0 = c * HALF + s * CPT + h * CPT2
        pltpu.sync_copy(src_hbm.at[pl.ds(t0, CPT2)], sidx_all)
        pltpu.sync_copy(dst_hbm.at[pl.ds(t0, CPT2)], didx_all)

        pltpu.async_copy(hs_hbm.at[sidx_all.at[0]], rows[0], gsem[0])

        def _grp(g, _):
            for b in range(2):
                k = g * 2 + b
                bn = 1 - b

                @pl.when(k > 0)
                def _():
                    pltpu.make_async_copy(rows[bn], acc_sp.at[pl.ds(0, C)],
                                          ssem[bn]).wait()

                @pl.when(k + 1 < CPT2)
                def _():
                    pltpu.async_copy(hs_hbm.at[sidx_all.at[k + 1]],
                                     rows[bn], gsem[bn])

                pltpu.make_async_copy(hs_hbm.at[pl.ds(0, C)], rows[b],
                                      gsem[b]).wait()
                pltpu.async_copy(rows[b], acc_sp.at[didx_all.at[k]],
                                 ssem[b], add=True)
            return 0

        lax.fori_loop(0, CPT2 // 2, _grp, 0)

        pltpu.make_async_copy(rows[1], acc_sp.at[pl.ds(0, C)],
                              ssem[1]).wait()

    plsc.subcore_barrier()

    def _wb(k, _):
        r0_ = s * ROWS_PER_TILE + k * C
        pltpu.sync_copy(acc_sp.at[pl.ds(r0_, C)], r0)
        pltpu.sync_copy(r0, out_hbm.at[pl.ds(c * NPAD + r0_, C)])
        return 0

    lax.fori_loop(0, ROWS_PER_TILE // C, _wb, 0)


@functools.partial(
    pl.kernel,
    out_type=jax.ShapeDtypeStruct((NC * NPAD, D), jnp.float32),
    mesh=_MESH,
    scratch_types=[
        pltpu.VMEM_SHARED((NPAD, D), jnp.float32),
        pltpu.VMEM((C, D), jnp.float32),
        pltpu.VMEM((C, D), jnp.float32),
        pltpu.VMEM((CPT2, C), jnp.int32),
        pltpu.VMEM((CPT2, C), jnp.int32),
        pltpu.SemaphoreType.DMA,
        pltpu.SemaphoreType.DMA,
        pltpu.SemaphoreType.DMA,
        pltpu.SemaphoreType.DMA,
    ],
    name="sc_edge_scatter",
)
def _sc_edge(hs_hbm, src_hbm, dst_hbm, out_hbm, acc_sp, r0, r1,
             sidx_all, didx_all, g0, g1, s0, s1):
    _edge_body(hs_hbm, src_hbm, dst_hbm, out_hbm, acc_sp, r0, r1,
               sidx_all, didx_all, g0, g1, s0, s1)


def _dinv_of(cnt_ref):
    deg = 1.0 + cnt_ref[0, :] + cnt_ref[1, :]
    return lax.rsqrt(deg)[:, None]


def _tc_mm_scale_body(x_ref, w_ref, cnt_ref, hs_ref):
    hs_ref[...] = jnp.dot(x_ref[...], w_ref[...],
                          preferred_element_type=jnp.float32) * _dinv_of(cnt_ref)


def _tc_mm_scale(x, w, cnt2):
    return pl.pallas_call(
        _tc_mm_scale_body,
        out_shape=jax.ShapeDtypeStruct((N, D), jnp.float32),
    )(x, w, cnt2)


def _tc_mid_body(acc_ref, hs_ref, cnt_ref, b_ref, w_ref, o_ref):
    dinv = _dinv_of(cnt_ref)
    g = dinv * (acc_ref[0:N, :] + acc_ref[NPAD:NPAD + N, :] + hs_ref[...]) \
        + b_ref[...]
    z = jnp.maximum(g, 0.0)
    h2 = jnp.dot(z, w_ref[...], preferred_element_type=jnp.float32)
    o_ref[...] = h2 * dinv


def _tc_mid(acc, hs1, cnt2, b1, w2):
    return pl.pallas_call(
        _tc_mid_body,
        out_shape=jax.ShapeDtypeStruct((N, D), jnp.float32),
    )(acc, hs1, cnt2, b1, w2)


def _tc_fin_body(acc_ref, hs_ref, cnt_ref, b_ref, wv_ref, bv_ref, wt_ref,
                 bt_ref, h_ref, xv_ref, xt_ref):
    dinv = _dinv_of(cnt_ref)
    h = dinv * (acc_ref[0:N, :] + acc_ref[NPAD:NPAD + N, :] + hs_ref[...]) \
        + b_ref[...]
    h_ref[...] = h
    xv_ref[...] = jnp.maximum(
        jnp.dot(h, wv_ref[...], preferred_element_type=jnp.float32)
        + bv_ref[...], 0.0)
    xt_ref[...] = jnp.maximum(
        jnp.dot(h, wt_ref[...], preferred_element_type=jnp.float32)
        + bt_ref[...], 0.0)


def _tc_fin(acc, hs2, cnt2, b2, wv, bv, wt, bt):
    return pl.pallas_call(
        _tc_fin_body,
        out_shape=(
            jax.ShapeDtypeStruct((N, D), jnp.float32),
            jax.ShapeDtypeStruct((N, D), jnp.float32),
            jax.ShapeDtypeStruct((N, D), jnp.float32),
        ),
    )(acc, hs2, cnt2, b2, wv, bv, wt, bt)


def kernel(x, edge_index, W1, b1, W2, b2, Wv, bv, Wt, bt):
    src = edge_index[0]
    dst = edge_index[1]
    npadded = EPAD - E
    pad_src = jnp.arange(npadded, dtype=jnp.int32) % N
    pad_dst = N + (jnp.arange(npadded, dtype=jnp.int32) % (NPAD - N))
    src_p = jnp.concatenate([src, pad_src])
    dst_p = jnp.concatenate([dst, pad_dst])
    src2 = src_p.reshape(NCHUNK, C)
    dst2 = dst_p.reshape(NCHUNK, C)
    b1r = b1.reshape(1, D)
    b2r = b2.reshape(1, D)
    bvr = bv.reshape(1, D)
    btr = bt.reshape(1, D)

    cnt2 = _sc_cnt(dst2).reshape(NC, N)
    hs1 = _tc_mm_scale(x, W1, cnt2)
    acc1 = _sc_edge(hs1, src2, dst2)
    hs2 = _tc_mid(acc1, hs1, cnt2, b1r, W2)
    acc2 = _sc_edge(hs2, src2, dst2)
    h, xv, xt = _tc_fin(acc2, hs2, cnt2, b2r, Wv, bvr, Wt, btr)
    return (h, xv, xt)


# speedup vs baseline: 1.0494x; 1.0459x over previous
import functools

import jax
import jax.numpy as jnp
from jax import lax
from jax.experimental import pallas as pl
from jax.experimental.pallas import tpu as pltpu
from jax.experimental.pallas import tpu_sc as plsc

N = 10000
D = 128
E = 320000

NC = 2
NS = 16
C = 64
NBUF = 4
NPAD = 10240
ROWS_PER_TILE = NPAD // NS

EPAD = 327680
NCHUNK = EPAD // C
HALF = NCHUNK // NC
CPT = HALF // NS
NSTAGE = 4
CPTS = CPT // NSTAGE

_MESH = plsc.VectorSubcoreMesh(core_axis_name="c", subcore_axis_name="s")


def _cnt_body(dst_hbm, out_hbm, cnt_sp, zbuf, ones_v, didx_all, csem):
    c = lax.axis_index("c")
    s = lax.axis_index("s")

    zeros16 = jnp.zeros((16,), jnp.float32)
    ones16 = jnp.ones((16,), jnp.float32)

    def _zb(i, _):
        zbuf[pl.ds(i * 16, 16)] = zeros16
        return 0

    lax.fori_loop(0, 2048 // 16, _zb, 0)

    def _ob(i, _):
        ones_v[pl.ds(i * 16, 16)] = ones16
        return 0

    lax.fori_loop(0, C // 16, _ob, 0)

    @pl.when(s < 5)
    def _():
        pltpu.sync_copy(zbuf, cnt_sp.at[pl.ds(s * 2048, 2048)])

    t0 = c * HALF + s * CPT
    pltpu.sync_copy(dst_hbm.at[pl.ds(t0, CPT)], didx_all)

    plsc.subcore_barrier()

    def _fire(k, _):
        pltpu.async_copy(ones_v, cnt_sp.at[didx_all.at[k]], csem, add=True)
        return 0

    lax.fori_loop(0, CPT, _fire, 0)

    def _drain(k, _):
        pltpu.make_async_copy(ones_v, cnt_sp.at[pl.ds(0, C)], csem).wait()
        return 0

    lax.fori_loop(0, CPT, _drain, 0)

    plsc.subcore_barrier()

    @pl.when(s < 10)
    def _():
        pltpu.sync_copy(cnt_sp.at[pl.ds(s * 1000, 1000)],
                        zbuf.at[pl.ds(0, 1000)])
        pltpu.sync_copy(zbuf.at[pl.ds(0, 1000)],
                        out_hbm.at[pl.ds(c * N + s * 1000, 1000)])


@functools.partial(
    pl.kernel,
    out_type=jax.ShapeDtypeStruct((NC * N,), jnp.float32),
    mesh=_MESH,
    scratch_types=[
        pltpu.VMEM_SHARED((NPAD,), jnp.float32),
        pltpu.VMEM((2048,), jnp.float32),
        pltpu.VMEM((C,), jnp.float32),
        pltpu.VMEM((CPT, C), jnp.int32),
        pltpu.SemaphoreType.DMA,
    ],
    name="sc_degree_count",
)
def _sc_cnt(dst_hbm, out_hbm, cnt_sp, zbuf, ones_v, didx_all, csem):
    _cnt_body(dst_hbm, out_hbm, cnt_sp, zbuf, ones_v, didx_all, csem)


def _edge_body(hs_hbm, src_hbm, dst_hbm, out_hbm, acc_sp, r0, r1, r2, r3,
               sidx_all, didx_all, g0, g1, g2, g3, s0, s1, s2, s3):
    c = lax.axis_index("c")
    s = lax.axis_index("s")
    rows = (r0, r1, r2, r3)
    gsem = (g0, g1, g2, g3)
    ssem = (s0, s1, s2, s3)

    zeros16 = jnp.zeros((16,), jnp.float32)

    def _zb(i, _):
        r0[i // 8, pl.ds((i % 8) * 16, 16)] = zeros16
        return 0

    lax.fori_loop(0, C * D // 16, _zb, 0)

    def _zc(k, _):
        pltpu.sync_copy(r0, acc_sp.at[pl.ds(s * ROWS_PER_TILE + k * C, C)])
        return 0

    lax.fori_loop(0, ROWS_PER_TILE // C, _zc, 0)

    plsc.subcore_barrier()

    for h in range(NSTAGE):
        t0 = c * HALF + s * CPT + h * CPTS
        pltpu.sync_copy(src_hbm.at[pl.ds(t0, CPTS)], sidx_all)
        pltpu.sync_copy(dst_hbm.at[pl.ds(t0, CPTS)], didx_all)

        for j in range(NBUF - 1):
            pltpu.async_copy(hs_hbm.at[sidx_all.at[j]], rows[j], gsem[j])

        def _grp(g, _):
            for b in range(NBUF):
                k = g * NBUF + b
                bg = (b + NBUF - 1) % NBUF

                @pl.when(k > 0)
                def _():
                    pltpu.make_async_copy(rows[bg], acc_sp.at[pl.ds(0, C)],
                                          ssem[bg]).wait()

                @pl.when(k + NBUF - 1 < CPTS)
                def _():
                    pltpu.async_copy(hs_hbm.at[sidx_all.at[k + NBUF - 1]],
                                     rows[bg], gsem[bg])

                pltpu.make_async_copy(hs_hbm.at[pl.ds(0, C)], rows[b],
                                      gsem[b]).wait()
                pltpu.async_copy(rows[b], acc_sp.at[didx_all.at[k]],
                                 ssem[b], add=True)
            return 0

        lax.fori_loop(0, CPTS // NBUF, _grp, 0)

        pltpu.make_async_copy(rows[(CPTS - 1) % NBUF], acc_sp.at[pl.ds(0, C)],
                              ssem[(CPTS - 1) % NBUF]).wait()

    plsc.subcore_barrier()

    def _wb(k, _):
        r0_ = s * ROWS_PER_TILE + k * C
        pltpu.sync_copy(acc_sp.at[pl.ds(r0_, C)], r0)
        pltpu.sync_copy(r0, out_hbm.at[pl.ds(c * NPAD + r0_, C)])
        return 0

    lax.fori_loop(0, ROWS_PER_TILE // C, _wb, 0)


@functools.partial(
    pl.kernel,
    out_type=jax.ShapeDtypeStruct((NC * NPAD, D), jnp.float32),
    mesh=_MESH,
    scratch_types=[
        pltpu.VMEM_SHARED((NPAD, D), jnp.float32),
        pltpu.VMEM((C, D), jnp.float32),
        pltpu.VMEM((C, D), jnp.float32),
        pltpu.VMEM((C, D), jnp.float32),
        pltpu.VMEM((C, D), jnp.float32),
        pltpu.VMEM((CPTS, C), jnp.int32),
        pltpu.VMEM((CPTS, C), jnp.int32),
        pltpu.SemaphoreType.DMA,
        pltpu.SemaphoreType.DMA,
        pltpu.SemaphoreType.DMA,
        pltpu.SemaphoreType.DMA,
        pltpu.SemaphoreType.DMA,
        pltpu.SemaphoreType.DMA,
        pltpu.SemaphoreType.DMA,
        pltpu.SemaphoreType.DMA,
    ],
    name="sc_edge_scatter",
)
def _sc_edge(hs_hbm, src_hbm, dst_hbm, out_hbm, acc_sp, r0, r1, r2, r3,
             sidx_all, didx_all, g0, g1, g2, g3, s0, s1, s2, s3):
    _edge_body(hs_hbm, src_hbm, dst_hbm, out_hbm, acc_sp, r0, r1, r2, r3,
               sidx_all, didx_all, g0, g1, g2, g3, s0, s1, s2, s3)


def _dinv_of(cnt_ref):
    deg = 1.0 + cnt_ref[0, :] + cnt_ref[1, :]
    return lax.rsqrt(deg)[:, None]


def _tc_mm_scale_body(x_ref, w_ref, cnt_ref, hs_ref):
    hs_ref[...] = jnp.dot(x_ref[...], w_ref[...],
                          preferred_element_type=jnp.float32) * _dinv_of(cnt_ref)


def _tc_mm_scale(x, w, cnt2):
    return pl.pallas_call(
        _tc_mm_scale_body,
        out_shape=jax.ShapeDtypeStruct((N, D), jnp.float32),
    )(x, w, cnt2)


def _tc_mid_body(acc_ref, hs_ref, cnt_ref, b_ref, w_ref, o_ref):
    dinv = _dinv_of(cnt_ref)
    g = dinv * (acc_ref[0:N, :] + acc_ref[NPAD:NPAD + N, :] + hs_ref[...]) \
        + b_ref[...]
    z = jnp.maximum(g, 0.0)
    h2 = jnp.dot(z, w_ref[...], preferred_element_type=jnp.float32)
    o_ref[...] = h2 * dinv


def _tc_mid(acc, hs1, cnt2, b1, w2):
    return pl.pallas_call(
        _tc_mid_body,
        out_shape=jax.ShapeDtypeStruct((N, D), jnp.float32),
    )(acc, hs1, cnt2, b1, w2)


def _tc_fin_body(acc_ref, hs_ref, cnt_ref, b_ref, wv_ref, bv_ref, wt_ref,
                 bt_ref, h_ref, xv_ref, xt_ref):
    dinv = _dinv_of(cnt_ref)
    h = dinv * (acc_ref[0:N, :] + acc_ref[NPAD:NPAD + N, :] + hs_ref[...]) \
        + b_ref[...]
    h_ref[...] = h
    xv_ref[...] = jnp.maximum(
        jnp.dot(h, wv_ref[...], preferred_element_type=jnp.float32)
        + bv_ref[...], 0.0)
    xt_ref[...] = jnp.maximum(
        jnp.dot(h, wt_ref[...], preferred_element_type=jnp.float32)
        + bt_ref[...], 0.0)


def _tc_fin(acc, hs2, cnt2, b2, wv, bv, wt, bt):
    return pl.pallas_call(
        _tc_fin_body,
        out_shape=(
            jax.ShapeDtypeStruct((N, D), jnp.float32),
            jax.ShapeDtypeStruct((N, D), jnp.float32),
            jax.ShapeDtypeStruct((N, D), jnp.float32),
        ),
    )(acc, hs2, cnt2, b2, wv, bv, wt, bt)


def kernel(x, edge_index, W1, b1, W2, b2, Wv, bv, Wt, bt):
    src = edge_index[0]
    dst = edge_index[1]
    npadded = EPAD - E
    pad_src = jnp.arange(npadded, dtype=jnp.int32) % N
    pad_dst = N + (jnp.arange(npadded, dtype=jnp.int32) % (NPAD - N))
    src_p = jnp.concatenate([src, pad_src])
    dst_p = jnp.concatenate([dst, pad_dst])
    src2 = src_p.reshape(NCHUNK, C)
    dst2 = dst_p.reshape(NCHUNK, C)
    b1r = b1.reshape(1, D)
    b2r = b2.reshape(1, D)
    bvr = bv.reshape(1, D)
    btr = bt.reshape(1, D)

    cnt2 = _sc_cnt(dst2).reshape(NC, N)
    hs1 = _tc_mm_scale(x, W1, cnt2)
    acc1 = _sc_edge(hs1, src2, dst2)
    hs2 = _tc_mid(acc1, hs1, cnt2, b1r, W2)
    acc2 = _sc_edge(hs2, src2, dst2)
    h, xv, xt = _tc_fin(acc2, hs2, cnt2, b2r, Wv, bvr, Wt, btr)
    return (h, xv, xt)
